# baseline (speedup 1.0000x reference)
import functools

import jax
import jax.numpy as jnp
from jax import lax
from jax.experimental import pallas as pl
from jax.experimental.pallas import tpu as pltpu
from jax.experimental.pallas import tpu_sc as plsc

_B = 2
_C = 192
_H = 224
_W = 224
_NIMG = _B * _C

_NC = 2
_NS = 16
_NW = _NC * _NS
_IPW = _NIMG // _NW


def _body(x_hbm, s_hbm, out_hbm, buf0, buf1, sem_l0, sem_l1, sem_s0, sem_s1):
    bufs = (buf0, buf1)
    sems_l = (sem_l0, sem_l1)
    sems_s = (sem_s0, sem_s1)
    wid = lax.axis_index("s") * _NC + lax.axis_index("c")
    base = wid * _IPW

    def coords(k):
        r = base + k
        b = jnp.where(r >= _C, 1, 0)
        c = r - b * _C
        return b, c

    def start_load(k):
        b, c = coords(k)
        is_x = c == 0

        @pl.when(is_x)
        def _():
            pltpu.make_async_copy(
                x_hbm.at[b, 0], bufs[k % 2], sems_l[k % 2]).start()

        @pl.when(jnp.logical_not(is_x))
        def _():
            pltpu.make_async_copy(
                s_hbm.at[b, c], bufs[k % 2], sems_l[k % 2]).start()

    def wait_load(k):
        pltpu.make_async_copy(
            s_hbm.at[0, 0], bufs[k % 2], sems_l[k % 2]).wait()

    stores = {}

    def start_store(k):
        b, c = coords(k)
        h = pltpu.make_async_copy(
            bufs[k % 2], out_hbm.at[b, c], sems_s[k % 2])
        h.start()
        stores[k] = h

    start_load(0)
    for k in range(_IPW):
        if k + 1 < _IPW:
            if k - 1 >= 0:
                stores[k - 1].wait()
            start_load(k + 1)
        wait_load(k)
        start_store(k)
    stores[_IPW - 2].wait()
    stores[_IPW - 1].wait()


def _tc_convert_body(s_ref, o_ref):
    o_ref[...] = jnp.transpose(s_ref[...], (0, 3, 1, 2))


def _tc_convert(s_t):
    return pl.pallas_call(
        _tc_convert_body,
        grid=(_B, _H // 8, 2, 2),
        in_specs=[pl.BlockSpec(
            (1, 8, 128, 128), lambda b, ht, wt, ct: (b, ht, wt, ct))],
        out_specs=pl.BlockSpec(
            (1, 128, 8, 128), lambda b, ht, wt, ct: (b, ct, ht, wt)),
        out_shape=jax.ShapeDtypeStruct((_B, _C, _H, _W), jnp.float32),
    )(s_t)


def kernel(x, shortcut_input):
    s_t = shortcut_input.transpose(0, 2, 3, 1)
    s_c = _tc_convert(s_t)
    mesh = plsc.VectorSubcoreMesh(
        core_axis_name="c", subcore_axis_name="s",
        num_cores=_NC, num_subcores=_NS)
    run = functools.partial(
        pl.kernel,
        mesh=mesh,
        out_type=jax.ShapeDtypeStruct((_B, _C, _H, _W), jnp.float32),
        scratch_types=[
            pltpu.VMEM((_H, _W), jnp.float32),
            pltpu.VMEM((_H, _W), jnp.float32),
            pltpu.SemaphoreType.DMA,
            pltpu.SemaphoreType.DMA,
            pltpu.SemaphoreType.DMA,
            pltpu.SemaphoreType.DMA,
        ],
        compiler_params=pltpu.CompilerParams(use_tc_tiling_on_sc=True),
    )(_body)
    return run(x, s_c)


# speedup vs baseline: 1.2170x; 1.2170x over previous
import functools

import jax
import jax.numpy as jnp
from jax import lax
from jax.experimental import pallas as pl
from jax.experimental.pallas import tpu as pltpu
from jax.experimental.pallas import tpu_sc as plsc

_B = 2
_C = 192
_H = 224
_W = 224
_NIMG = _B * _C

_NC = 2
_NS = 16
_NW = _NC * _NS
_IPW = _NIMG // _NW


def _body(x_hbm, s_hbm, out_hbm, buf0, buf1, sem_l0, sem_l1, sem_s0, sem_s1):
    bufs = (buf0, buf1)
    sems_l = (sem_l0, sem_l1)
    sems_s = (sem_s0, sem_s1)
    wid = lax.axis_index("s") * _NC + lax.axis_index("c")
    base = wid * _IPW

    def coords(k):
        r = base + k
        b = jnp.where(r >= _C, 1, 0)
        c = r - b * _C
        return b, c

    def start_load(k):
        b, c = coords(k)
        is_x = c == 0

        @pl.when(is_x)
        def _():
            pltpu.make_async_copy(
                x_hbm.at[b, 0], bufs[k % 2], sems_l[k % 2]).start()

        @pl.when(jnp.logical_not(is_x))
        def _():
            pltpu.make_async_copy(
                s_hbm.at[b, c], bufs[k % 2], sems_l[k % 2]).start()

    def wait_load(k):
        pltpu.make_async_copy(
            s_hbm.at[0, 0], bufs[k % 2], sems_l[k % 2]).wait()

    stores = {}

    def start_store(k):
        b, c = coords(k)
        h = pltpu.make_async_copy(
            bufs[k % 2], out_hbm.at[b, c], sems_s[k % 2])
        h.start()
        stores[k] = h

    start_load(0)
    for k in range(_IPW):
        if k + 1 < _IPW:
            if k - 1 >= 0:
                stores[k - 1].wait()
            start_load(k + 1)
        wait_load(k)
        start_store(k)
    stores[_IPW - 2].wait()
    stores[_IPW - 1].wait()


def _tc_convert_body(s_ref, o_ref):
    o_ref[...] = jnp.transpose(s_ref[...], (0, 3, 1, 2))


def _tc_convert(s_t):
    return pl.pallas_call(
        _tc_convert_body,
        grid=(_B, _H // 8, 2),
        in_specs=[pl.BlockSpec(
            (1, 8, _W, 128), lambda b, ht, ct: (b, ht, 0, ct))],
        out_specs=pl.BlockSpec(
            (1, 128, 8, _W), lambda b, ht, ct: (b, ct, ht, 0)),
        out_shape=jax.ShapeDtypeStruct((_B, _C, _H, _W), jnp.float32),
        compiler_params=pltpu.CompilerParams(
            dimension_semantics=("parallel", "parallel", "parallel")),
    )(s_t)


def kernel(x, shortcut_input):
    s_t = shortcut_input.transpose(0, 2, 3, 1)
    s_c = _tc_convert(s_t)
    mesh = plsc.VectorSubcoreMesh(
        core_axis_name="c", subcore_axis_name="s",
        num_cores=_NC, num_subcores=_NS)
    run = functools.partial(
        pl.kernel,
        mesh=mesh,
        out_type=jax.ShapeDtypeStruct((_B, _C, _H, _W), jnp.float32),
        scratch_types=[
            pltpu.VMEM((_H, _W), jnp.float32),
            pltpu.VMEM((_H, _W), jnp.float32),
            pltpu.SemaphoreType.DMA,
            pltpu.SemaphoreType.DMA,
            pltpu.SemaphoreType.DMA,
            pltpu.SemaphoreType.DMA,
        ],
        compiler_params=pltpu.CompilerParams(use_tc_tiling_on_sc=True),
    )(_body)
    return run(x, s_c)


# speedup vs baseline: 1.4974x; 1.2304x over previous
import functools

import jax
import jax.numpy as jnp
from jax import lax
from jax.experimental import pallas as pl
from jax.experimental.pallas import tpu as pltpu
from jax.experimental.pallas import tpu_sc as plsc

_B = 2
_C = 192
_H = 224
_W = 224
_NIMG = _B * _C

_NC = 2
_NS = 16
_NW = _NC * _NS
_IPW = _NIMG // _NW


def _body(x_hbm, s_hbm, out_hbm, buf0, buf1, sem_l0, sem_l1, sem_s0, sem_s1):
    bufs = (buf0, buf1)
    sems_l = (sem_l0, sem_l1)
    sems_s = (sem_s0, sem_s1)
    wid = lax.axis_index("s") * _NC + lax.axis_index("c")
    base = wid * _IPW

    def coords(k):
        r = base + k
        b = jnp.where(r >= _C, 1, 0)
        c = r - b * _C
        return b, c

    def start_load(k):
        b, c = coords(k)
        is_x = c == 0

        @pl.when(is_x)
        def _():
            pltpu.make_async_copy(
                x_hbm.at[b, 0], bufs[k % 2], sems_l[k % 2]).start()

        @pl.when(jnp.logical_not(is_x))
        def _():
            pltpu.make_async_copy(
                s_hbm.at[b, c], bufs[k % 2], sems_l[k % 2]).start()

    def wait_load(k):
        pltpu.make_async_copy(
            s_hbm.at[0, 0], bufs[k % 2], sems_l[k % 2]).wait()

    stores = {}

    def start_store(k):
        b, c = coords(k)
        h = pltpu.make_async_copy(
            bufs[k % 2], out_hbm.at[b, c], sems_s[k % 2])
        h.start()
        stores[k] = h

    start_load(0)
    for k in range(_IPW):
        if k + 1 < _IPW:
            if k - 1 >= 0:
                stores[k - 1].wait()
            start_load(k + 1)
        wait_load(k)
        start_store(k)
    stores[_IPW - 2].wait()
    stores[_IPW - 1].wait()


def _tc_convert_body(s_ref, o_ref):
    o_ref[...] = jnp.transpose(s_ref[...], (0, 3, 1, 2))


def _tc_convert(s_t):
    return pl.pallas_call(
        _tc_convert_body,
        grid=(_B, _H // 16, 2),
        in_specs=[pl.BlockSpec(
            (1, 16, _W, 128), lambda b, ht, ct: (b, ht, 0, ct))],
        out_specs=pl.BlockSpec(
            (1, 128, 16, _W), lambda b, ht, ct: (b, ct, ht, 0)),
        out_shape=jax.ShapeDtypeStruct((_B, _C, _H, _W), jnp.float32),
        compiler_params=pltpu.CompilerParams(
            dimension_semantics=("parallel", "parallel", "parallel")),
    )(s_t)


def kernel(x, shortcut_input):
    s_t = shortcut_input.transpose(0, 2, 3, 1)
    s_c = _tc_convert(s_t)
    mesh = plsc.VectorSubcoreMesh(
        core_axis_name="c", subcore_axis_name="s",
        num_cores=_NC, num_subcores=_NS)
    run = functools.partial(
        pl.kernel,
        mesh=mesh,
        out_type=jax.ShapeDtypeStruct((_B, _C, _H, _W), jnp.float32),
        scratch_types=[
            pltpu.VMEM((_H, _W), jnp.float32),
            pltpu.VMEM((_H, _W), jnp.float32),
            pltpu.SemaphoreType.DMA,
            pltpu.SemaphoreType.DMA,
            pltpu.SemaphoreType.DMA,
            pltpu.SemaphoreType.DMA,
        ],
        compiler_params=pltpu.CompilerParams(use_tc_tiling_on_sc=True),
    )(_body)
    return run(x, s_c)


# speedup vs baseline: 1.6639x; 1.1112x over previous
import functools

import jax
import jax.numpy as jnp
from jax import lax
from jax.experimental import pallas as pl
from jax.experimental.pallas import tpu as pltpu
from jax.experimental.pallas import tpu_sc as plsc

_B = 2
_C = 192
_H = 224
_W = 224
_NIMG = _B * _C

_NC = 2
_NS = 16
_NW = _NC * _NS
_IPW = _NIMG // _NW


def _body(x_hbm, s_hbm, out_hbm, buf0, buf1, sem_l0, sem_l1, sem_s0, sem_s1):
    bufs = (buf0, buf1)
    sems_l = (sem_l0, sem_l1)
    sems_s = (sem_s0, sem_s1)
    wid = lax.axis_index("s") * _NC + lax.axis_index("c")
    base = wid * _IPW

    def coords(k):
        r = base + k
        b = jnp.where(r >= _C, 1, 0)
        c = r - b * _C
        return b, c

    def start_load(k):
        b, c = coords(k)
        is_x = c == 0

        @pl.when(is_x)
        def _():
            pltpu.make_async_copy(
                x_hbm.at[b, 0], bufs[k % 2], sems_l[k % 2]).start()

        @pl.when(jnp.logical_not(is_x))
        def _():
            pltpu.make_async_copy(
                s_hbm.at[b, c], bufs[k % 2], sems_l[k % 2]).start()

    def wait_load(k):
        pltpu.make_async_copy(
            s_hbm.at[0, 0], bufs[k % 2], sems_l[k % 2]).wait()

    stores = {}

    def start_store(k):
        b, c = coords(k)
        h = pltpu.make_async_copy(
            bufs[k % 2], out_hbm.at[b, c], sems_s[k % 2])
        h.start()
        stores[k] = h

    start_load(0)
    for k in range(_IPW):
        if k + 1 < _IPW:
            if k - 1 >= 0:
                stores[k - 1].wait()
            start_load(k + 1)
        wait_load(k)
        start_store(k)
    stores[_IPW - 2].wait()
    stores[_IPW - 1].wait()


def _tc_convert_body(s_ref, o_ref):
    o_ref[...] = jnp.transpose(s_ref[...], (0, 3, 1, 2))


def _tc_convert(s_t):
    return pl.pallas_call(
        _tc_convert_body,
        grid=(_B, _H // 32, 2),
        in_specs=[pl.BlockSpec(
            (1, 32, _W, 128), lambda b, ht, ct: (b, ht, 0, ct))],
        out_specs=pl.BlockSpec(
            (1, 128, 32, _W), lambda b, ht, ct: (b, ct, ht, 0)),
        out_shape=jax.ShapeDtypeStruct((_B, _C, _H, _W), jnp.float32),
        compiler_params=pltpu.CompilerParams(
            dimension_semantics=("parallel", "parallel", "parallel")),
    )(s_t)


def kernel(x, shortcut_input):
    s_t = shortcut_input.transpose(0, 2, 3, 1)
    s_c = _tc_convert(s_t)
    mesh = plsc.VectorSubcoreMesh(
        core_axis_name="c", subcore_axis_name="s",
        num_cores=_NC, num_subcores=_NS)
    run = functools.partial(
        pl.kernel,
        mesh=mesh,
        out_type=jax.ShapeDtypeStruct((_B, _C, _H, _W), jnp.float32),
        scratch_types=[
            pltpu.VMEM((_H, _W), jnp.float32),
            pltpu.VMEM((_H, _W), jnp.float32),
            pltpu.SemaphoreType.DMA,
            pltpu.SemaphoreType.DMA,
            pltpu.SemaphoreType.DMA,
            pltpu.SemaphoreType.DMA,
        ],
        compiler_params=pltpu.CompilerParams(use_tc_tiling_on_sc=True),
    )(_body)
    return run(x, s_c)


# speedup vs baseline: 1.7586x; 1.0569x over previous
import functools

import jax
import jax.numpy as jnp
from jax import lax
from jax.experimental import pallas as pl
from jax.experimental.pallas import tpu as pltpu
from jax.experimental.pallas import tpu_sc as plsc

_B = 2
_C = 192
_H = 224
_W = 224
_NIMG = _B * _C

_NC = 2
_NS = 16
_NW = _NC * _NS
_IPW = _NIMG // _NW


def _body(x_hbm, s_hbm, out_hbm, buf0, buf1, sem_l0, sem_l1, sem_s0, sem_s1):
    bufs = (buf0, buf1)
    sems_l = (sem_l0, sem_l1)
    sems_s = (sem_s0, sem_s1)
    wid = lax.axis_index("s") * _NC + lax.axis_index("c")
    base = wid * _IPW

    def coords(k):
        r = base + k
        b = jnp.where(r >= _C, 1, 0)
        c = r - b * _C
        return b, c

    def start_load(k):
        b, c = coords(k)
        is_x = c == 0

        @pl.when(is_x)
        def _():
            pltpu.make_async_copy(
                x_hbm.at[b, 0], bufs[k % 2], sems_l[k % 2]).start()

        @pl.when(jnp.logical_not(is_x))
        def _():
            pltpu.make_async_copy(
                s_hbm.at[b, c], bufs[k % 2], sems_l[k % 2]).start()

    def wait_load(k):
        pltpu.make_async_copy(
            s_hbm.at[0, 0], bufs[k % 2], sems_l[k % 2]).wait()

    stores = {}

    def start_store(k):
        b, c = coords(k)
        h = pltpu.make_async_copy(
            bufs[k % 2], out_hbm.at[b, c], sems_s[k % 2])
        h.start()
        stores[k] = h

    start_load(0)
    for k in range(_IPW):
        if k + 1 < _IPW:
            if k - 1 >= 0:
                stores[k - 1].wait()
            start_load(k + 1)
        wait_load(k)
        start_store(k)
    stores[_IPW - 2].wait()
    stores[_IPW - 1].wait()


def _tc_convert_body(s_ref, o_ref):
    o_ref[...] = jnp.transpose(s_ref[...], (0, 3, 1, 2))


def _tc_convert(s_t):
    return pl.pallas_call(
        _tc_convert_body,
        grid=(_B, _H // 56, 2),
        in_specs=[pl.BlockSpec(
            (1, 56, _W, 128), lambda b, ht, ct: (b, ht, 0, ct))],
        out_specs=pl.BlockSpec(
            (1, 128, 56, _W), lambda b, ht, ct: (b, ct, ht, 0)),
        out_shape=jax.ShapeDtypeStruct((_B, _C, _H, _W), jnp.float32),
        compiler_params=pltpu.CompilerParams(
            dimension_semantics=("parallel", "parallel", "parallel")),
    )(s_t)


def kernel(x, shortcut_input):
    s_t = shortcut_input.transpose(0, 2, 3, 1)
    s_c = _tc_convert(s_t)
    mesh = plsc.VectorSubcoreMesh(
        core_axis_name="c", subcore_axis_name="s",
        num_cores=_NC, num_subcores=_NS)
    run = functools.partial(
        pl.kernel,
        mesh=mesh,
        out_type=jax.ShapeDtypeStruct((_B, _C, _H, _W), jnp.float32),
        scratch_types=[
            pltpu.VMEM((_H, _W), jnp.float32),
            pltpu.VMEM((_H, _W), jnp.float32),
            pltpu.SemaphoreType.DMA,
            pltpu.SemaphoreType.DMA,
            pltpu.SemaphoreType.DMA,
            pltpu.SemaphoreType.DMA,
        ],
        compiler_params=pltpu.CompilerParams(use_tc_tiling_on_sc=True),
    )(_body)
    return run(x, s_c)
